# baseline (speedup 1.0000x reference)
import functools

import jax
import jax.numpy as jnp
from jax import lax
from jax.experimental import pallas as pl
from jax.experimental.pallas import tpu as pltpu
from jax.experimental.pallas import tpu_sc as plsc

NUM_EMB = 1_000_000
D = 64
B = 16384

_info = plsc.get_sparse_core_info()
_NC, _NS, _L = _info.num_cores, _info.num_subcores, _info.num_lanes
_NW = _NC * _NS
_BPW = B // _NW
_CHUNKS = D // _L

_mesh = plsc.VectorSubcoreMesh(core_axis_name="c", subcore_axis_name="s")


@functools.partial(
    pl.kernel,
    mesh=_mesh,
    out_type=jax.ShapeDtypeStruct((B, D), jnp.float32),
    scratch_types=[
        pltpu.VMEM((_BPW,), jnp.int32),
        pltpu.VMEM((_BPW, D), jnp.float32),
        pltpu.VMEM((D,), jnp.float32),
        pltpu.SemaphoreType.DMA,
    ],
    compiler_params=pltpu.CompilerParams(use_tc_tiling_on_sc=False),
)
def _kb_lookup(idx_hbm, table_hbm, rel_hbm, out_hbm, idx_v, rows_v, rel_v, sem):
    wid = lax.axis_index("s") * _NC + lax.axis_index("c")
    base = wid * _BPW

    pltpu.sync_copy(idx_hbm.at[pl.ds(base, _BPW)], idx_v)
    pltpu.sync_copy(rel_hbm, rel_v)
    pltpu.async_copy(table_hbm.at[idx_v], rows_v, sem).wait()

    rel_c = [rel_v[pl.ds(c * _L, _L)] for c in range(_CHUNKS)]

    def row_body(i, carry):
        for c in range(_CHUNKS):
            sl = pl.ds(c * _L, _L)
            rows_v[i, sl] = rows_v[i, sl] + rel_c[c]
        return carry

    lax.fori_loop(0, _BPW, row_body, 0)

    pltpu.sync_copy(rows_v, out_hbm.at[pl.ds(base, _BPW)])


def kernel(entity_idx, entity_table, relation_embedding):
    return _kb_lookup(
        entity_idx.astype(jnp.int32), entity_table, relation_embedding
    )
